# baseline (speedup 1.0000x reference)
import functools

import jax
import jax.numpy as jnp
from jax.experimental import pallas as pl
from jax.experimental.pallas import tpu as pltpu

_N = 4096
_D = 2048
_R = 64
_E = 8
_ALPHA = 2.0
_C = (_E + 1) * _R
_BN = 512


def _moe_lora_body(lab_ref, x_ref, a_spec_ref, a_gen_ref, b_spec_ref,
                   b_gen_ref, o_ref, a_scr, b_scr):
    i = pl.program_id(0)

    @pl.when(i == 0)
    def _init():
        a_scr[0:_E * _R, :] = a_spec_ref[...]
        a_scr[_E * _R:_C, :] = a_gen_ref[...]
        rr = jax.lax.broadcasted_iota(jnp.int32, (_R, _R), 0)
        cc = jax.lax.broadcasted_iota(jnp.int32, (_R, _R), 1)
        eye = (rr == cc).astype(jnp.float32)
        for e in range(_E):
            b_scr[e * _R:(e + 1) * _R, :] = jax.lax.dot_general(
                eye, b_spec_ref[e], (((1,), (1,)), ((), ())),
                preferred_element_type=jnp.float32)
        b_scr[_E * _R:_C, :] = jax.lax.dot_general(
            eye, b_gen_ref[...], (((1,), (1,)), ((), ())),
            preferred_element_type=jnp.float32)

    x = x_ref[...]
    h = jax.lax.dot_general(
        x, a_scr[...], (((1,), (1,)), ((), ())),
        preferred_element_type=jnp.float32,
    )
    lab = lab_ref[...]
    col = jax.lax.broadcasted_iota(jnp.int32, h.shape, 1)
    keep = (col // _R == lab) | (col >= _E * _R)
    h = jnp.where(keep, h * _ALPHA, 0.0)
    out = jax.lax.dot_general(
        h, b_scr[...], (((1,), (0,)), ((), ())),
        preferred_element_type=jnp.float32,
    )
    row = jax.lax.broadcasted_iota(jnp.int32, out.shape, 0) + i * _BN
    o_ref[...] = jnp.where(row == _N - 1, 0.0, out)


@functools.partial(jax.jit, static_argnames=())
def kernel(x, label, weight, A_gen, B_gen, A_spec, B_spec):
    del weight
    lab = label.astype(jnp.int32).reshape(_N, 1)
    a_spec = A_spec.reshape(_E * _R, _D)
    return pl.pallas_call(
        _moe_lora_body,
        grid=(_N // _BN,),
        in_specs=[
            pl.BlockSpec((_BN, 1), lambda i: (i, 0)),
            pl.BlockSpec((_BN, _D), lambda i: (i, 0)),
            pl.BlockSpec((_E * _R, _D), lambda i: (0, 0)),
            pl.BlockSpec((_R, _D), lambda i: (0, 0)),
            pl.BlockSpec((_E, _D, _R), lambda i: (0, 0, 0)),
            pl.BlockSpec((_D, _R), lambda i: (0, 0)),
        ],
        out_specs=pl.BlockSpec((_BN, _D), lambda i: (i, 0)),
        out_shape=jax.ShapeDtypeStruct((_N, _D), jnp.float32),
        scratch_shapes=[
            pltpu.VMEM((_C, _D), jnp.float32),
            pltpu.VMEM((_C, _D), jnp.float32),
        ],
    )(lab, x, a_spec, A_gen, B_spec, B_gen)
